# baseline (speedup 1.0000x reference)
import functools

import jax
import jax.numpy as jnp
from jax import lax
from jax.experimental import pallas as pl
from jax.experimental.pallas import tpu as pltpu
from jax.experimental.pallas import tpu_sc as plsc

VOCAB = 100000
D = 768
BATCH = 4
SEQ = 4096
LANES = 16

NC = 2
NS = 16
NW = NC * NS

POS_PER_W = SEQ // NW
CHUNK = 32
N_CHUNKS = POS_PER_W // CHUNK
VECS_PER_CHUNK = CHUNK * D // LANES


def _pe_table():
    pos = jnp.arange(SEQ, dtype=jnp.float32)[:, None]
    i = jnp.arange(0, D, 2, dtype=jnp.float32)
    div = jnp.power(10000.0, i / D)
    pe = jnp.zeros((SEQ, D), dtype=jnp.float32)
    pe = pe.at[:, 0::2].set(jnp.sin(pos / div))
    pe = pe.at[:, 1::2].set(jnp.cos(pos / div))
    return pe


def _sc_body(x_hbm, pe_hbm, tab_hbm, out_hbm, idx_v, pe_v, rows_v, gsem):
    wid = lax.axis_index("s") * NC + lax.axis_index("c")
    pos0 = wid * POS_PER_W

    for c in range(N_CHUNKS):
        cbase = pos0 + c * CHUNK
        pltpu.sync_copy(pe_hbm.at[pl.ds(cbase, CHUNK)], pe_v)
        for b in range(BATCH):
            fbase = b * SEQ + cbase
            pltpu.sync_copy(x_hbm.at[pl.ds(fbase, CHUNK)], idx_v)
            pltpu.async_copy(tab_hbm.at[idx_v], rows_v, gsem).wait()

            def add_body(k, _):
                i = k // (D // LANES)
                j = (k % (D // LANES)) * LANES
                rows_v[i, pl.ds(j, LANES)] = (
                    rows_v[i, pl.ds(j, LANES)] + pe_v[i, pl.ds(j, LANES)]
                )
                return 0

            lax.fori_loop(0, VECS_PER_CHUNK, add_body, 0)
            pltpu.sync_copy(rows_v, out_hbm.at[pl.ds(fbase, CHUNK)])


@jax.jit
def kernel(x, tok_table):
    pe = _pe_table()
    x_flat = x.reshape(BATCH * SEQ).astype(jnp.int32)

    mesh = plsc.VectorSubcoreMesh(core_axis_name="c", subcore_axis_name="s")
    run = pl.kernel(
        _sc_body,
        out_type=jax.ShapeDtypeStruct((BATCH * SEQ, D), jnp.float32),
        mesh=mesh,
        scratch_types=[
            pltpu.VMEM((CHUNK,), jnp.int32),
            pltpu.VMEM((CHUNK, D), jnp.float32),
            pltpu.VMEM((CHUNK, D), jnp.float32),
            pltpu.SemaphoreType.DMA,
        ],
    )
    out = run(x_flat, pe, tok_table)
    return out.reshape(BATCH, SEQ, D)


# speedup vs baseline: 1.8111x; 1.8111x over previous
import jax
import jax.numpy as jnp
from jax import lax
from jax.experimental import pallas as pl
from jax.experimental.pallas import tpu as pltpu
from jax.experimental.pallas import tpu_sc as plsc

VOCAB = 100000
D = 768
BATCH = 4
SEQ = 4096
LANES = 16
D_VECS = D // LANES

NC = 2
NS = 16
NW = NC * NS

POS_PER_W = SEQ // NW
CHUNK = 32
N_CHUNKS = POS_PER_W // CHUNK
ROUNDS = N_CHUNKS * BATCH
NB = 3
NPE = 2


def _pe_table():
    pos = jnp.arange(SEQ, dtype=jnp.float32)[:, None]
    i = jnp.arange(0, D, 2, dtype=jnp.float32)
    div = jnp.power(10000.0, i / D)
    pe = jnp.zeros((SEQ, D), dtype=jnp.float32)
    pe = pe.at[:, 0::2].set(jnp.sin(pos / div))
    pe = pe.at[:, 1::2].set(jnp.cos(pos / div))
    return pe


def _sc_body(x_hbm, pe_hbm, tab_hbm, out_hbm, idx_v, rows, pe_v,
             pe_sem, g_sem, st_sem):
    wid = lax.axis_index("s") * NC + lax.axis_index("c")
    pos0 = wid * POS_PER_W

    for b in range(BATCH):
        pltpu.sync_copy(x_hbm.at[pl.ds(b * SEQ + pos0, POS_PER_W)], idx_v.at[b])

    def cb(r):
        return r // BATCH, r % BATCH

    def issue_pe(c):
        return pltpu.async_copy(
            pe_hbm.at[pl.ds(pos0 + c * CHUNK, CHUNK)], pe_v[c % NPE],
            pe_sem[c % NPE])

    def issue_g(r):
        c, b = cb(r)
        buf = r % NB
        return pltpu.async_copy(
            tab_hbm.at[idx_v.at[b, pl.ds(c * CHUNK, CHUNK)]],
            rows[buf], g_sem[buf])

    def issue_st(r):
        c, b = cb(r)
        buf = r % NB
        return pltpu.async_copy(
            rows[buf], out_hbm.at[pl.ds(b * SEQ + pos0 + c * CHUNK, CHUNK)],
            st_sem[buf])

    def add_pe(r):
        c, _ = cb(r)
        rbuf, pbuf = rows[r % NB], pe_v[c % NPE]

        def body(i, _):
            for j in range(D_VECS):
                sl = pl.ds(j * LANES, LANES)
                plsc.addupdate(rbuf.at[i, sl], pbuf[i, sl])
            return 0

        lax.fori_loop(0, CHUNK, body, 0)

    d_pe, d_g, d_st = {}, {}, {}
    d_pe[0] = issue_pe(0)
    if N_CHUNKS > 1:
        d_pe[1] = issue_pe(1)
    d_g[0] = issue_g(0)
    for r in range(ROUNDS):
        c, b = cb(r)
        if r + 1 < ROUNDS:
            if r + 1 >= NB:
                d_st[r + 1 - NB].wait()
            d_g[r + 1] = issue_g(r + 1)
        d_g[r].wait()
        if b == 0:
            d_pe[c].wait()
        add_pe(r)
        d_st[r] = issue_st(r)
        if b == BATCH - 1 and c + 2 < N_CHUNKS:
            d_pe[c + 2] = issue_pe(c + 2)
    for r in range(max(0, ROUNDS - NB), ROUNDS):
        if r in d_st:
            d_st[r].wait()


@jax.jit
def kernel(x, tok_table):
    pe = _pe_table()
    x_flat = x.reshape(BATCH * SEQ).astype(jnp.int32)

    mesh = plsc.VectorSubcoreMesh(core_axis_name="c", subcore_axis_name="s")
    run = pl.kernel(
        _sc_body,
        out_type=jax.ShapeDtypeStruct((BATCH * SEQ, D), jnp.float32),
        mesh=mesh,
        scratch_types=[
            pltpu.VMEM((BATCH, POS_PER_W), jnp.int32),
            [pltpu.VMEM((CHUNK, D), jnp.float32) for _ in range(NB)],
            [pltpu.VMEM((CHUNK, D), jnp.float32) for _ in range(NPE)],
            [pltpu.SemaphoreType.DMA for _ in range(NPE)],
            [pltpu.SemaphoreType.DMA for _ in range(NB)],
            [pltpu.SemaphoreType.DMA for _ in range(NB)],
        ],
    )
    out = run(x_flat, pe, tok_table)
    return out.reshape(BATCH, SEQ, D)


# speedup vs baseline: 3.2732x; 1.8073x over previous
import jax
import jax.numpy as jnp
import numpy as np
from jax import lax
from jax.experimental import pallas as pl
from jax.experimental.pallas import tpu as pltpu
from jax.experimental.pallas import tpu_sc as plsc

VOCAB = 100000
D = 768
BATCH = 4
SEQ = 4096
LANES = 16
D_VECS = D // LANES

NC = 2
NS = 16
NW = NC * NS

POS_PER_W = SEQ // NW
CHUNK = 32
N_CHUNKS = POS_PER_W // CHUNK
ROUNDS = N_CHUNKS * BATCH
NB = 3
NPE = 2


def _pe_table():
    pos = np.arange(SEQ, dtype=np.float32)[:, None]
    i = np.arange(0, D, 2, dtype=np.float32)
    div = np.power(np.float32(10000.0), i / np.float32(D))
    pe = np.zeros((SEQ, D), dtype=np.float32)
    pe[:, 0::2] = np.sin(pos / div, dtype=np.float32)
    pe[:, 1::2] = np.cos(pos / div, dtype=np.float32)
    return jnp.asarray(pe)


def _sc_body(x_hbm, pe_hbm, tab_hbm, out_hbm, idx_v, rows, pe_v,
             pe_sem, g_sem, st_sem):
    wid = lax.axis_index("s") * NC + lax.axis_index("c")
    pos0 = wid * POS_PER_W

    for b in range(BATCH):
        pltpu.sync_copy(x_hbm.at[pl.ds(b * SEQ + pos0, POS_PER_W)], idx_v.at[b])

    def cb(r):
        return r // BATCH, r % BATCH

    def issue_pe(c):
        return pltpu.async_copy(
            pe_hbm.at[pl.ds(pos0 + c * CHUNK, CHUNK)], pe_v[c % NPE],
            pe_sem[c % NPE])

    def issue_g(r):
        c, b = cb(r)
        buf = r % NB
        return pltpu.async_copy(
            tab_hbm.at[idx_v.at[b, pl.ds(c * CHUNK, CHUNK)]],
            rows[buf], g_sem[buf])

    def issue_st(r):
        c, b = cb(r)
        buf = r % NB
        return pltpu.async_copy(
            rows[buf], out_hbm.at[pl.ds(b * SEQ + pos0 + c * CHUNK, CHUNK)],
            st_sem[buf])

    def add_pe(r):
        c, _ = cb(r)
        rbuf, pbuf = rows[r % NB], pe_v[c % NPE]

        def body(i, _):
            for j in range(D_VECS):
                sl = pl.ds(j * LANES, LANES)
                plsc.addupdate(rbuf.at[i, sl], pbuf[i, sl])
            return 0

        lax.fori_loop(0, CHUNK, body, 0)

    d_pe, d_g, d_st = {}, {}, {}
    d_pe[0] = issue_pe(0)
    if N_CHUNKS > 1:
        d_pe[1] = issue_pe(1)
    d_g[0] = issue_g(0)
    for r in range(ROUNDS):
        c, b = cb(r)
        if r + 1 < ROUNDS:
            if r + 1 >= NB:
                d_st[r + 1 - NB].wait()
            d_g[r + 1] = issue_g(r + 1)
        d_g[r].wait()
        if b == 0:
            d_pe[c].wait()
        add_pe(r)
        d_st[r] = issue_st(r)
        if b == BATCH - 1 and c + 2 < N_CHUNKS:
            d_pe[c + 2] = issue_pe(c + 2)
    for r in range(max(0, ROUNDS - NB), ROUNDS):
        if r in d_st:
            d_st[r].wait()


@jax.jit
def kernel(x, tok_table):
    pe = _pe_table()
    x_flat = x.reshape(BATCH * SEQ).astype(jnp.int32)

    mesh = plsc.VectorSubcoreMesh(core_axis_name="c", subcore_axis_name="s")
    run = pl.kernel(
        _sc_body,
        out_type=jax.ShapeDtypeStruct((BATCH * SEQ, D), jnp.float32),
        mesh=mesh,
        scratch_types=[
            pltpu.VMEM((BATCH, POS_PER_W), jnp.int32),
            [pltpu.VMEM((CHUNK, D), jnp.float32) for _ in range(NB)],
            [pltpu.VMEM((CHUNK, D), jnp.float32) for _ in range(NPE)],
            [pltpu.SemaphoreType.DMA for _ in range(NPE)],
            [pltpu.SemaphoreType.DMA for _ in range(NB)],
            [pltpu.SemaphoreType.DMA for _ in range(NB)],
        ],
    )
    out = run(x_flat, pe, tok_table)
    return out.reshape(BATCH, SEQ, D)
